# Initial kernel scaffold; baseline (speedup 1.0000x reference)
import jax
import jax.numpy as jnp
from jax.experimental import pallas as pl


def kernel(sub_x, x, edge_index, up_idx, W_mix, b_mix, W_res, b_res, gamma_res, beta_res):
    raise NotImplementedError("write your pallas kernel here")


# speedup vs baseline: 1.7003x; 1.7003x over previous
import functools

import jax
import jax.numpy as jnp
from jax import lax
from jax.experimental import pallas as pl
from jax.experimental.pallas import tpu as pltpu
from jax.experimental.pallas import tpu_sc as plsc

N = 10000
NSUB = 2500
E = 320000
C = 128

NC, NS, L = 2, 16, 16
CG = 8
NCG = C // CG
EH = E // 2
CHUNK = 2000
NCHUNK = EH // CHUNK

_MESH = plsc.VectorSubcoreMesh(core_axis_name="c", subcore_axis_name="s",
                               num_cores=NC, num_subcores=NS)

GW = 25
GROWS = N // GW


@functools.partial(
    pl.kernel,
    out_type=jax.ShapeDtypeStruct((N, 2 * C), jnp.float32),
    mesh=_MESH,
    scratch_types=[
        pltpu.VMEM((GROWS,), jnp.int32),
        pltpu.VMEM((GROWS, 2 * C), jnp.float32),
        pltpu.SemaphoreType.DMA,
    ],
)
def _up_gather(tab_hbm, idx_hbm, out_hbm, idx_v, rows_v, sem):
    wid = lax.axis_index("c") * NS + lax.axis_index("s")

    @pl.when(wid < GW)
    def _():
        base = wid * GROWS
        pltpu.sync_copy(idx_hbm.at[pl.ds(base, GROWS)], idx_v)
        pltpu.async_copy(tab_hbm.at[idx_v], rows_v, sem).wait()
        pltpu.sync_copy(rows_v, out_hbm.at[pl.ds(base, GROWS)])


@functools.partial(
    pl.kernel,
    out_type=jax.ShapeDtypeStruct((2, NCG, N * CG), jnp.float32),
    mesh=_MESH,
    compiler_params=pltpu.CompilerParams(needs_layout_passes=False, use_tc_tiling_on_sc=False),
    scratch_types=[
        pltpu.VMEM((N * CG,), jnp.float32),
        pltpu.VMEM((CHUNK,), jnp.int32),
        pltpu.VMEM((CHUNK,), jnp.int32),
        pltpu.VMEM((CHUNK, CG), jnp.float32),
        pltpu.SemaphoreType.DMA,
    ],
)
def _seg_max(ag_hbm, src_hbm, dst_hbm, out_hbm, acc, src_v, dst_v, row_v, sem):
    half = lax.axis_index("c")
    cg = lax.axis_index("s")

    iota = lax.iota(jnp.int32, L)
    lane8 = iota & 7
    hi = (iota >> 3) & 1
    lo = 1 - hi
    neg_inf = jnp.full((L,), -jnp.inf, jnp.float32)

    def init(k, _):
        acc[pl.ds(k * L, L)] = neg_inf
        return 0

    lax.fori_loop(0, N * CG // L, init, 0)

    def chunk_body(ci, _):
        base = half * EH + ci * CHUNK
        pltpu.sync_copy(src_hbm.at[pl.ds(base, CHUNK)], src_v)
        pltpu.sync_copy(dst_hbm.at[pl.ds(base, CHUNK)], dst_v)
        pltpu.async_copy(ag_hbm.at[cg].at[src_v], row_v, sem).wait()

        def edge_body(i, _):
            i0 = 2 * i
            rows = i0 + hi
            rows_sw = i0 + lo
            d = plsc.load_gather(dst_v, [rows])
            d_sw = plsc.load_gather(dst_v, [rows_sw])
            v = plsc.load_gather(row_v, [rows, lane8])
            v_sw = plsc.load_gather(row_v, [rows_sw, lane8])
            v = jnp.where(d == d_sw, jnp.maximum(v, v_sw), v)
            aidx = d * CG + lane8
            old = plsc.load_gather(acc, [aidx])
            plsc.store_scatter(acc, [aidx], jnp.maximum(old, v))
            return 0

        lax.fori_loop(0, CHUNK // 2, edge_body, 0)
        return 0

    lax.fori_loop(0, NCHUNK, chunk_body, 0)
    pltpu.sync_copy(acc, out_hbm.at[half, cg])


def _segment_max(a, src, dst):
    ag = a.reshape(N, NCG, CG).transpose(1, 0, 2)
    part = _seg_max(ag, src, dst)
    agg = jnp.max(part, axis=0).reshape(NCG, N, CG)
    return agg.transpose(1, 0, 2).reshape(N, C)


def kernel(sub_x, x, edge_index, up_idx, W_mix, b_mix, W_res, b_res,
           gamma_res, beta_res):
    src = edge_index[0]
    dst = edge_index[1]

    W1m, W2m = W_mix[:2 * C], W_mix[2 * C:]
    Dm = W1m - W2m
    sub_proj = jnp.concatenate([sub_x @ W2m[C:], sub_x @ Dm[C:]], axis=1)
    up = _up_gather(sub_proj, up_idx)
    A1 = x @ W2m[:C] + up[:, :C]
    B1 = x @ Dm[:C] + up[:, C:2 * C] + b_mix

    s1 = _segment_max(A1, src, dst) + B1
    h = jnp.where(jnp.isfinite(s1), s1, 0.0)

    W1r, W2r = W_res[:C], W_res[C:]
    A2 = h @ W2r
    B2 = h @ (W1r - W2r) + b_res
    s2 = _segment_max(A2, src, dst) + B2
    h_res = jnp.where(jnp.isfinite(s2), s2, 0.0)

    mu = jnp.mean(h_res, axis=0, keepdims=True)
    var = jnp.var(h_res, axis=0, keepdims=True)
    hn = (h_res - mu) / jnp.sqrt(var + 1e-5) * gamma_res + beta_res
    return jax.nn.relu(hn) + h


# speedup vs baseline: 1.8225x; 1.0718x over previous
import functools

import jax
import jax.numpy as jnp
from jax import lax
from jax.experimental import pallas as pl
from jax.experimental.pallas import tpu as pltpu
from jax.experimental.pallas import tpu_sc as plsc

N = 10000
NSUB = 2500
E = 320000
C = 128

NC, NS, L = 2, 16, 16
CG = 8
NCG = C // CG
EH = E // 2
CHUNK = 2000
NCHUNK = EH // CHUNK
UNROLL = 4

_MESH = plsc.VectorSubcoreMesh(core_axis_name="c", subcore_axis_name="s",
                               num_cores=NC, num_subcores=NS)

GW = 25
GROWS = N // GW


@functools.partial(
    pl.kernel,
    out_type=jax.ShapeDtypeStruct((N, 2 * C), jnp.float32),
    mesh=_MESH,
    scratch_types=[
        pltpu.VMEM((GROWS,), jnp.int32),
        pltpu.VMEM((GROWS, 2 * C), jnp.float32),
        pltpu.SemaphoreType.DMA,
    ],
)
def _up_gather(tab_hbm, idx_hbm, out_hbm, idx_v, rows_v, sem):
    wid = lax.axis_index("c") * NS + lax.axis_index("s")

    @pl.when(wid < GW)
    def _():
        base = wid * GROWS
        pltpu.sync_copy(idx_hbm.at[pl.ds(base, GROWS)], idx_v)
        pltpu.async_copy(tab_hbm.at[idx_v], rows_v, sem).wait()
        pltpu.sync_copy(rows_v, out_hbm.at[pl.ds(base, GROWS)])


@functools.partial(
    pl.kernel,
    out_type=jax.ShapeDtypeStruct((2, NCG, N * CG), jnp.float32),
    mesh=_MESH,
    compiler_params=pltpu.CompilerParams(needs_layout_passes=False, use_tc_tiling_on_sc=False),
    scratch_types=[
        pltpu.VMEM((N * CG,), jnp.float32),
        pltpu.VMEM((CHUNK,), jnp.int32),
        pltpu.VMEM((CHUNK,), jnp.int32),
        pltpu.VMEM((CHUNK, CG), jnp.float32),
        pltpu.SemaphoreType.DMA,
    ],
)
def _seg_max(ag_hbm, src_hbm, dst_hbm, out_hbm, acc, src_v, dst_v, row_v, sem):
    half = lax.axis_index("c")
    cg = lax.axis_index("s")

    iota = lax.iota(jnp.int32, L)
    lane8 = iota & 7
    hi = (iota >> 3) & 1
    lo = 1 - hi
    neg_inf = jnp.full((L,), -jnp.inf, jnp.float32)

    def init(k, _):
        acc[pl.ds(k * L, L)] = neg_inf
        return 0

    lax.fori_loop(0, N * CG // L, init, 0)

    def chunk_body(ci, _):
        base = half * EH + ci * CHUNK
        pltpu.sync_copy(src_hbm.at[pl.ds(base, CHUNK)], src_v)
        pltpu.sync_copy(dst_hbm.at[pl.ds(base, CHUNK)], dst_v)
        pltpu.async_copy(ag_hbm.at[cg].at[src_v], row_v, sem).wait()

        def edge_body(i, _):
            for j in range(UNROLL):
                i0 = 2 * (UNROLL * i + j)
                rows = i0 + hi
                rows_sw = i0 + lo
                d = plsc.load_gather(dst_v, [rows])
                d_sw = plsc.load_gather(dst_v, [rows_sw])
                v = plsc.load_gather(row_v, [rows, lane8])
                v_sw = plsc.load_gather(row_v, [rows_sw, lane8])
                v = jnp.where(d == d_sw, jnp.maximum(v, v_sw), v)
                aidx = d * CG + lane8
                old = plsc.load_gather(acc, [aidx])
                plsc.store_scatter(acc, [aidx], jnp.maximum(old, v))
            return 0

        lax.fori_loop(0, CHUNK // (2 * UNROLL), edge_body, 0)
        return 0

    lax.fori_loop(0, NCHUNK, chunk_body, 0)
    pltpu.sync_copy(acc, out_hbm.at[half, cg])


def _segment_max(a, src, dst):
    ag = a.reshape(N, NCG, CG).transpose(1, 0, 2)
    part = _seg_max(ag, src, dst)
    agg = jnp.max(part, axis=0).reshape(NCG, N, CG)
    return agg.transpose(1, 0, 2).reshape(N, C)


def kernel(sub_x, x, edge_index, up_idx, W_mix, b_mix, W_res, b_res,
           gamma_res, beta_res):
    src = edge_index[0]
    dst = edge_index[1]

    W1m, W2m = W_mix[:2 * C], W_mix[2 * C:]
    Dm = W1m - W2m
    sub_proj = jnp.concatenate([sub_x @ W2m[C:], sub_x @ Dm[C:]], axis=1)
    up = _up_gather(sub_proj, up_idx)
    A1 = x @ W2m[:C] + up[:, :C]
    B1 = x @ Dm[:C] + up[:, C:2 * C] + b_mix

    s1 = _segment_max(A1, src, dst) + B1
    h = jnp.where(jnp.isfinite(s1), s1, 0.0)

    W1r, W2r = W_res[:C], W_res[C:]
    A2 = h @ W2r
    B2 = h @ (W1r - W2r) + b_res
    s2 = _segment_max(A2, src, dst) + B2
    h_res = jnp.where(jnp.isfinite(s2), s2, 0.0)

    mu = jnp.mean(h_res, axis=0, keepdims=True)
    var = jnp.var(h_res, axis=0, keepdims=True)
    hn = (h_res - mu) / jnp.sqrt(var + 1e-5) * gamma_res + beta_res
    return jax.nn.relu(hn) + h
